# baseline (speedup 1.0000x reference)
import functools

import jax
import jax.numpy as jnp
from jax import lax
from jax.experimental import pallas as pl
from jax.experimental.pallas import tpu as pltpu
from jax.experimental.pallas import tpu_sc as plsc

_NC = 2
_NS = 16
_NW = _NC * _NS

_BATCH = 1024
_SEQ = 200
_NTOK = 10
_DIM = 64
_OUT_S = _NTOK + _SEQ
_VOCAB = 1000000

_QUART = 1 << 18
_BLK_T = 4096
_NBT = _BATCH // 128


def _tc_pack(wte_t):
    nb = _QUART // _BLK_T
    last = _VOCAB // _BLK_T

    def enc(ref):
        b = lax.bitcast_convert_type(ref[...].T, jnp.int32)
        lo = b[:, 0:32]
        hi = b[:, 32:64]
        return ((((hi + 0x8000) >> 16) << 16)
                | (((lo + 0x8000) >> 16) & 0xFFFF))

    def body(w0, w1, w2, w3, out_ref):
        out_ref[...] = jnp.concatenate(
            [enc(w0), enc(w1), enc(w2), enc(w3)], axis=1)

    return pl.pallas_call(
        body,
        grid=(nb,),
        in_specs=[
            pl.BlockSpec((_DIM, _BLK_T),
                         lambda k, q=q: (0, jnp.minimum(k + q * 64, last)))
            for q in range(4)
        ],
        out_specs=pl.BlockSpec((_BLK_T, 2 * _DIM), lambda k: (k, 0)),
        out_shape=jax.ShapeDtypeStruct((_QUART, 2 * _DIM), jnp.int32),
    )(wte_t, wte_t, wte_t, wte_t)


def _make_sc_kernel():
    mesh = plsc.VectorSubcoreMesh(core_axis_name="c", subcore_axis_name="s")

    @functools.partial(
        pl.kernel,
        out_type=jax.ShapeDtypeStruct((_SEQ, _NBT, 128, 128), jnp.int32),
        mesh=mesh,
        scratch_types=[
            pltpu.VMEM((_BATCH,), jnp.int32),
            pltpu.VMEM((_BATCH,), jnp.int32),
            pltpu.VMEM((4, 128, 128), jnp.int32),
            pltpu.SemaphoreType.DMA,
            pltpu.SemaphoreType.DMA,
            pltpu.SemaphoreType.DMA,
            pltpu.SemaphoreType.DMA,
            pltpu.SemaphoreType.DMA,
            pltpu.SemaphoreType.DMA,
            pltpu.SemaphoreType.DMA,
            pltpu.SemaphoreType.DMA,
        ],
        compiler_params=pltpu.CompilerParams(needs_layout_passes=False),
    )
    def sc_gather(wtp_hbm, tok_hbm, g_hbm,
                  tokrow, idxa, rows,
                  gsem0, gsem1, gsem2, gsem3, ssem0, ssem1, ssem2, ssem3):
        w = lax.axis_index("s") * _NC + lax.axis_index("c")
        gsems = (gsem0, gsem1, gsem2, gsem3)
        ssems = (ssem0, ssem1, ssem2, ssem3)
        n_s = (_SEQ - 1 - w) // _NW + 1

        @pl.loop(0, n_s)
        def _souter(si):
            s_tok = si * _NW + w
            pltpu.sync_copy(tok_hbm.at[pl.ds(s_tok * _BATCH, _BATCH)], tokrow)

            @pl.loop(0, 64, unroll=4)
            def _prep(g):
                idxa[pl.ds(g * 16, 16)] = \
                    tokrow[pl.ds(g * 16, 16)] & (_QUART - 1)

            def issue_gather(bt, slot):
                return pltpu.async_copy(
                    wtp_hbm.at[idxa.at[pl.ds(bt * 128, 128)]],
                    rows.at[slot], gsems[slot])

            g_desc = [issue_gather(j, j) for j in range(4)]
            s_desc = [None] * 4
            for bt in range(_NBT):
                slot = bt & 3
                g_desc[slot].wait()
                s_desc[slot] = pltpu.async_copy(
                    rows.at[slot], g_hbm.at[s_tok, bt], ssems[slot])
                if bt + 4 < _NBT:
                    s_desc[slot].wait()
                    g_desc[slot] = issue_gather(bt + 4, slot)
            for j in range(4):
                s_desc[j].wait()

    return sc_gather


try:
    _SC_GATHER = _make_sc_kernel()
except ValueError:
    _SC_GATHER = None


def _tc_out(g5, tok_flat, learned):
    def body(g_ref, tok_ref, lrn_ref, out_ref):
        sp = pl.program_id(0)

        @pl.when(sp < _NTOK)
        def _learned():
            row = lrn_ref[pl.ds(jnp.minimum(sp, _NTOK - 1), 1), :]
            out_ref[...] = jnp.broadcast_to(
                row.reshape(_DIM, 1), (_DIM, 128)).reshape(1, 8, 1, 8, 128)

        @pl.when(sp >= _NTOK)
        def _gather():
            g = g_ref[...].reshape(128, 128)
            q = (tok_ref[...].reshape(128) >> 18)[:, None]
            z01 = jnp.where((q & 1) == 1, g[:, 32:64], g[:, 0:32])
            z23 = jnp.where((q & 1) == 1, g[:, 96:128], g[:, 64:96])
            z = jnp.where((q & 2) == 2, z23, z01)
            flo = lax.bitcast_convert_type(z << 16, jnp.float32)
            fhi = lax.bitcast_convert_type(z & jnp.int32(-65536), jnp.float32)
            y = jnp.concatenate([flo.T, fhi.T], axis=0)
            out_ref[...] = y.reshape(1, 8, 1, 8, 128)

    return pl.pallas_call(
        body,
        grid=(_OUT_S, _NBT),
        in_specs=[
            pl.BlockSpec((1, 1, 128, 128),
                         lambda s, bt: (jnp.maximum(s - _NTOK, 0), bt, 0, 0)),
            pl.BlockSpec((1, 1, 1, 128),
                         lambda s, bt: (jnp.maximum(s - _NTOK, 0), bt, 0, 0)),
            pl.BlockSpec((_NTOK, _DIM), lambda s, bt: (0, 0)),
        ],
        out_specs=pl.BlockSpec((1, 8, 1, 8, 128),
                               lambda s, bt: (s, 0, bt, 0, 0)),
        out_shape=jax.ShapeDtypeStruct((_OUT_S, 8, _NBT, 8, 128),
                                       jnp.float32),
    )(g5, tok_flat, learned)


def kernel(tokens, wte_weight, learned_embedding):
    wtp = _tc_pack(wte_weight.T)
    tok2 = tokens.astype(jnp.int32).T.reshape(_SEQ, _NBT, 128)
    g5 = _SC_GATHER(wtp, tok2.reshape(-1))
    out5 = _tc_out(g5, tok2.reshape(_SEQ, _NBT, 1, 128), learned_embedding)
    return (out5.transpose(2, 4, 0, 1, 3)
            .reshape(_BATCH, _OUT_S, _DIM))


# speedup vs baseline: 1.8002x; 1.8002x over previous
import functools

import jax
import jax.numpy as jnp
from jax import lax
from jax.experimental import pallas as pl
from jax.experimental.pallas import tpu as pltpu
from jax.experimental.pallas import tpu_sc as plsc

_NC = 2
_NS = 16
_NW = _NC * _NS

_BATCH = 1024
_SEQ = 200
_NTOK = 10
_DIM = 64
_OUT_S = _NTOK + _SEQ
_VOCAB = 1000000

_QUART = 1 << 18
_BLK_T = 4096
_NBT = _BATCH // 128


def _tc_pack(wte_t):
    nb = _QUART // _BLK_T
    last = _VOCAB // _BLK_T

    def enc(ref):
        b = lax.bitcast_convert_type(ref[...].T, jnp.int32)
        lo = b[:, 0:32]
        hi = b[:, 32:64]
        return ((((hi + 0x8000) >> 16) << 16)
                | (((lo + 0x8000) >> 16) & 0xFFFF))

    def body(w0, w1, w2, w3, out_ref):
        out_ref[...] = jnp.concatenate(
            [enc(w0), enc(w1), enc(w2), enc(w3)], axis=1)

    return pl.pallas_call(
        body,
        grid=(nb,),
        in_specs=[
            pl.BlockSpec((_DIM, _BLK_T),
                         lambda k, q=q: (0, jnp.minimum(k + q * 64, last)))
            for q in range(4)
        ],
        out_specs=pl.BlockSpec((_BLK_T, 2 * _DIM), lambda k: (k, 0)),
        out_shape=jax.ShapeDtypeStruct((_QUART, 2 * _DIM), jnp.int32),
    )(wte_t, wte_t, wte_t, wte_t)


def _make_sc_kernel():
    mesh = plsc.VectorSubcoreMesh(core_axis_name="c", subcore_axis_name="s")

    @functools.partial(
        pl.kernel,
        out_type=jax.ShapeDtypeStruct((_SEQ, _NBT, 128, 128), jnp.int32),
        mesh=mesh,
        scratch_types=[
            pltpu.VMEM((_BATCH,), jnp.int32),
            pltpu.VMEM((_BATCH,), jnp.int32),
            pltpu.VMEM((4, 128, 128), jnp.int32),
            pltpu.SemaphoreType.DMA,
            pltpu.SemaphoreType.DMA,
            pltpu.SemaphoreType.DMA,
            pltpu.SemaphoreType.DMA,
            pltpu.SemaphoreType.DMA,
            pltpu.SemaphoreType.DMA,
            pltpu.SemaphoreType.DMA,
            pltpu.SemaphoreType.DMA,
        ],
        compiler_params=pltpu.CompilerParams(needs_layout_passes=False),
    )
    def sc_gather(wtp_hbm, tok_hbm, g_hbm,
                  tokrow, idxa, rows,
                  gsem0, gsem1, gsem2, gsem3, ssem0, ssem1, ssem2, ssem3):
        w = lax.axis_index("s") * _NC + lax.axis_index("c")
        gsems = (gsem0, gsem1, gsem2, gsem3)
        ssems = (ssem0, ssem1, ssem2, ssem3)
        n_s = (_SEQ - 1 - w) // _NW + 1

        @pl.loop(0, n_s)
        def _souter(si):
            s_tok = si * _NW + w
            pltpu.sync_copy(tok_hbm.at[pl.ds(s_tok * _BATCH, _BATCH)], tokrow)

            @pl.loop(0, 64, unroll=4)
            def _prep(g):
                idxa[pl.ds(g * 16, 16)] = \
                    tokrow[pl.ds(g * 16, 16)] & (_QUART - 1)

            def issue_gather(bt, slot):
                return pltpu.async_copy(
                    wtp_hbm.at[idxa.at[pl.ds(bt * 128, 128)]],
                    rows.at[slot], gsems[slot])

            g_desc = [issue_gather(j, j) for j in range(4)]
            s_desc = [None] * 4
            for bt in range(_NBT):
                slot = bt & 3
                g_desc[slot].wait()
                s_desc[slot] = pltpu.async_copy(
                    rows.at[slot], g_hbm.at[s_tok, bt], ssems[slot])
                if bt + 4 < _NBT:
                    s_desc[slot].wait()
                    g_desc[slot] = issue_gather(bt + 4, slot)
            for j in range(4):
                s_desc[j].wait()

    return sc_gather


try:
    _SC_GATHER = _make_sc_kernel()
except ValueError:
    _SC_GATHER = None


def _tc_out(g5, tok_flat, learned):
    def body(g_ref, tok_ref, lrn_ref, out_ref):
        sp = pl.program_id(0)

        @pl.when(sp < _NTOK)
        def _learned():
            row = lrn_ref[pl.ds(jnp.minimum(sp, _NTOK - 1), 1), :]
            blk = jnp.broadcast_to(
                row.reshape(_DIM, 1), (_DIM, 128)).reshape(1, 8, 1, 8, 128)
            for bt in range(_NBT):
                out_ref[:, :, bt, :, :] = blk.reshape(1, 8, 8, 128)

        @pl.when(sp >= _NTOK)
        def _gather():
            for bt in range(_NBT):
                g = g_ref[0, bt]
                q = (tok_ref[0, bt, 0] >> 18)[:, None]
                z01 = jnp.where((q & 1) == 1, g[:, 32:64], g[:, 0:32])
                z23 = jnp.where((q & 1) == 1, g[:, 96:128], g[:, 64:96])
                z = jnp.where((q & 2) == 2, z23, z01)
                flo = lax.bitcast_convert_type(z << 16, jnp.float32)
                fhi = lax.bitcast_convert_type(
                    z & jnp.int32(-65536), jnp.float32)
                y = jnp.concatenate([flo.T, fhi.T], axis=0)
                out_ref[:, :, bt, :, :] = y.reshape(1, 8, 8, 128)

    return pl.pallas_call(
        body,
        grid=(_OUT_S,),
        in_specs=[
            pl.BlockSpec((1, _NBT, 128, 128),
                         lambda s: (jnp.maximum(s - _NTOK, 0), 0, 0, 0)),
            pl.BlockSpec((1, _NBT, 1, 128),
                         lambda s: (jnp.maximum(s - _NTOK, 0), 0, 0, 0)),
            pl.BlockSpec((_NTOK, _DIM), lambda s: (0, 0)),
        ],
        out_specs=pl.BlockSpec((1, 8, _NBT, 8, 128),
                               lambda s: (s, 0, 0, 0, 0)),
        out_shape=jax.ShapeDtypeStruct((_OUT_S, 8, _NBT, 8, 128),
                                       jnp.float32),
    )(g5, tok_flat, learned)


def kernel(tokens, wte_weight, learned_embedding):
    wtp = _tc_pack(wte_weight.T)
    tok2 = tokens.astype(jnp.int32).T.reshape(_SEQ, _NBT, 128)
    g5 = _SC_GATHER(wtp, tok2.reshape(-1))
    out5 = _tc_out(g5, tok2.reshape(_SEQ, _NBT, 1, 128), learned_embedding)
    return (out5.transpose(2, 4, 0, 1, 3)
            .reshape(_BATCH, _OUT_S, _DIM))


# speedup vs baseline: 2.1560x; 1.1976x over previous
import functools

import jax
import jax.numpy as jnp
from jax import lax
from jax.experimental import pallas as pl
from jax.experimental.pallas import tpu as pltpu
from jax.experimental.pallas import tpu_sc as plsc

_NC = 2
_NS = 16
_NW = _NC * _NS

_BATCH = 1024
_SEQ = 200
_NTOK = 10
_DIM = 64
_OUT_S = _NTOK + _SEQ
_VOCAB = 1000000

_QUART = 1 << 18
_BLK_T = 4096
_NBT = _BATCH // 128


def _tc_pack(wte_t):
    nb = _QUART // _BLK_T
    last = _VOCAB // _BLK_T

    def enc(ref):
        b = lax.bitcast_convert_type(ref[...], jnp.int32)
        lo = b[0:32, :]
        hi = b[32:64, :]
        w = ((((hi + 0x8000) >> 16) << 16)
             | (((lo + 0x8000) >> 16) & 0xFFFF))
        return w.T

    def body(w0, w1, w2, w3, out_ref):
        out_ref[...] = jnp.concatenate(
            [enc(w0), enc(w1), enc(w2), enc(w3)], axis=1)

    return pl.pallas_call(
        body,
        grid=(nb,),
        in_specs=[
            pl.BlockSpec((_DIM, _BLK_T),
                         lambda k, q=q: (0, jnp.minimum(k + q * 64, last)))
            for q in range(4)
        ],
        out_specs=pl.BlockSpec((_BLK_T, 2 * _DIM), lambda k: (k, 0)),
        out_shape=jax.ShapeDtypeStruct((_QUART, 2 * _DIM), jnp.int32),
    )(wte_t, wte_t, wte_t, wte_t)


def _make_sc_kernel():
    mesh = plsc.VectorSubcoreMesh(core_axis_name="c", subcore_axis_name="s")

    @functools.partial(
        pl.kernel,
        out_type=jax.ShapeDtypeStruct((_SEQ, _NBT, 128, 128), jnp.int32),
        mesh=mesh,
        scratch_types=[
            pltpu.VMEM((_BATCH,), jnp.int32),
            pltpu.VMEM((_BATCH,), jnp.int32),
            pltpu.VMEM((4, 128, 128), jnp.int32),
            pltpu.SemaphoreType.DMA,
            pltpu.SemaphoreType.DMA,
            pltpu.SemaphoreType.DMA,
            pltpu.SemaphoreType.DMA,
            pltpu.SemaphoreType.DMA,
            pltpu.SemaphoreType.DMA,
            pltpu.SemaphoreType.DMA,
            pltpu.SemaphoreType.DMA,
        ],
        compiler_params=pltpu.CompilerParams(needs_layout_passes=False),
    )
    def sc_gather(wtp_hbm, tok_hbm, g_hbm,
                  tokrow, idxa, rows,
                  gsem0, gsem1, gsem2, gsem3, ssem0, ssem1, ssem2, ssem3):
        w = lax.axis_index("s") * _NC + lax.axis_index("c")
        gsems = (gsem0, gsem1, gsem2, gsem3)
        ssems = (ssem0, ssem1, ssem2, ssem3)
        n_s = (_SEQ - 1 - w) // _NW + 1

        @pl.loop(0, n_s)
        def _souter(si):
            s_tok = si * _NW + w
            pltpu.sync_copy(tok_hbm.at[pl.ds(s_tok * _BATCH, _BATCH)], tokrow)

            @pl.loop(0, 64, unroll=4)
            def _prep(g):
                idxa[pl.ds(g * 16, 16)] = \
                    tokrow[pl.ds(g * 16, 16)] & (_QUART - 1)

            def issue_gather(bt, slot):
                return pltpu.async_copy(
                    wtp_hbm.at[idxa.at[pl.ds(bt * 128, 128)]],
                    rows.at[slot], gsems[slot])

            g_desc = [issue_gather(j, j) for j in range(4)]
            s_desc = [None] * 4
            for bt in range(_NBT):
                slot = bt & 3
                g_desc[slot].wait()
                s_desc[slot] = pltpu.async_copy(
                    rows.at[slot], g_hbm.at[s_tok, bt], ssems[slot])
                if bt + 4 < _NBT:
                    s_desc[slot].wait()
                    g_desc[slot] = issue_gather(bt + 4, slot)
            for j in range(4):
                s_desc[j].wait()

    return sc_gather


try:
    _SC_GATHER = _make_sc_kernel()
except ValueError:
    _SC_GATHER = None


def _tc_out(g5, tok_flat, learned):
    def body(g_ref, tok_ref, lrn_ref, out_ref):
        sp = pl.program_id(0)

        @pl.when(sp < _NTOK)
        def _learned():
            row = lrn_ref[pl.ds(jnp.minimum(sp, _NTOK - 1), 1), :]
            blk = jnp.broadcast_to(
                row.reshape(1, _DIM, 1, 1), (1, _DIM, _NBT, 128))
            out_ref[...] = (blk.reshape(1, 8, 8, _NBT, 128)
                            .transpose(0, 1, 3, 2, 4))

        @pl.when(sp >= _NTOK)
        def _gather():
            ys = []
            for bt in range(_NBT):
                g = g_ref[0, bt]
                q = (tok_ref[0, bt, 0] >> 18)[:, None]
                z01 = jnp.where((q & 1) == 1, g[:, 32:64], g[:, 0:32])
                z23 = jnp.where((q & 1) == 1, g[:, 96:128], g[:, 64:96])
                z = jnp.where((q & 2) == 2, z23, z01)
                flo = lax.bitcast_convert_type(z << 16, jnp.float32)
                fhi = lax.bitcast_convert_type(
                    z & jnp.int32(-65536), jnp.float32)
                y = jnp.concatenate([flo.T, fhi.T], axis=0)
                ys.append(y.reshape(1, 8, 8, 128))
            big = jnp.concatenate(ys, axis=0)
            out_ref[...] = big.transpose(1, 0, 2, 3).reshape(1, 8, 8, 8, 128)

    return pl.pallas_call(
        body,
        grid=(_OUT_S,),
        in_specs=[
            pl.BlockSpec((1, _NBT, 128, 128),
                         lambda s: (jnp.maximum(s - _NTOK, 0), 0, 0, 0)),
            pl.BlockSpec((1, _NBT, 1, 128),
                         lambda s: (jnp.maximum(s - _NTOK, 0), 0, 0, 0)),
            pl.BlockSpec((_NTOK, _DIM), lambda s: (0, 0)),
        ],
        out_specs=pl.BlockSpec((1, 8, _NBT, 8, 128),
                               lambda s: (s, 0, 0, 0, 0)),
        out_shape=jax.ShapeDtypeStruct((_OUT_S, 8, _NBT, 8, 128),
                                       jnp.float32),
    )(g5, tok_flat, learned)


def kernel(tokens, wte_weight, learned_embedding):
    wtp = _tc_pack(wte_weight.T)
    tok2 = tokens.astype(jnp.int32).T.reshape(_SEQ, _NBT, 128)
    g5 = _SC_GATHER(wtp, tok2.reshape(-1))
    out5 = _tc_out(g5, tok2.reshape(_SEQ, _NBT, 1, 128), learned_embedding)
    return (out5.transpose(2, 4, 0, 1, 3)
            .reshape(_BATCH, _OUT_S, _DIM))


# speedup vs baseline: 2.1712x; 1.0070x over previous
import functools

import jax
import jax.numpy as jnp
from jax import lax
from jax.experimental import pallas as pl
from jax.experimental.pallas import tpu as pltpu
from jax.experimental.pallas import tpu_sc as plsc

_NC = 2
_NS = 16
_NW = _NC * _NS

_BATCH = 1024
_SEQ = 200
_NTOK = 10
_DIM = 64
_OUT_S = _NTOK + _SEQ
_VOCAB = 1000000

_QUART = 1 << 18
_BLK_T = 8192
_NBT = _BATCH // 128


def _tc_pack(wte_t):
    nb = _QUART // _BLK_T
    last = _VOCAB // _BLK_T

    def enc(ref):
        b = lax.bitcast_convert_type(ref[...], jnp.int32)
        lo = b[0:32, :]
        hi = b[32:64, :]
        w = ((((hi + 0x8000) >> 16) << 16)
             | (((lo + 0x8000) >> 16) & 0xFFFF))
        return w.T

    def body(w0, w1, w2, w3, out_ref):
        out_ref[...] = jnp.concatenate(
            [enc(w0), enc(w1), enc(w2), enc(w3)], axis=1)

    return pl.pallas_call(
        body,
        grid=(nb,),
        in_specs=[
            pl.BlockSpec((_DIM, _BLK_T),
                         lambda k, q=q: (0, jnp.minimum(k + q * 64, last)))
            for q in range(4)
        ],
        out_specs=pl.BlockSpec((_BLK_T, 2 * _DIM), lambda k: (k, 0)),
        out_shape=jax.ShapeDtypeStruct((_QUART, 2 * _DIM), jnp.int32),
    )(wte_t, wte_t, wte_t, wte_t)


def _make_sc_kernel():
    mesh = plsc.VectorSubcoreMesh(core_axis_name="c", subcore_axis_name="s")

    @functools.partial(
        pl.kernel,
        out_type=jax.ShapeDtypeStruct((_SEQ, _NBT, 128, 128), jnp.int32),
        mesh=mesh,
        scratch_types=[
            pltpu.VMEM((_BATCH,), jnp.int32),
            pltpu.VMEM((_BATCH,), jnp.int32),
            pltpu.VMEM((4, 128, 128), jnp.int32),
            pltpu.SemaphoreType.DMA,
            pltpu.SemaphoreType.DMA,
            pltpu.SemaphoreType.DMA,
            pltpu.SemaphoreType.DMA,
            pltpu.SemaphoreType.DMA,
            pltpu.SemaphoreType.DMA,
            pltpu.SemaphoreType.DMA,
            pltpu.SemaphoreType.DMA,
        ],
        compiler_params=pltpu.CompilerParams(needs_layout_passes=False),
    )
    def sc_gather(wtp_hbm, tok_hbm, g_hbm,
                  tokrow, idxa, rows,
                  gsem0, gsem1, gsem2, gsem3, ssem0, ssem1, ssem2, ssem3):
        w = lax.axis_index("s") * _NC + lax.axis_index("c")
        gsems = (gsem0, gsem1, gsem2, gsem3)
        ssems = (ssem0, ssem1, ssem2, ssem3)
        n_s = (_SEQ - 1 - w) // _NW + 1

        @pl.loop(0, n_s)
        def _souter(si):
            s_tok = si * _NW + w
            pltpu.sync_copy(tok_hbm.at[pl.ds(s_tok * _BATCH, _BATCH)], tokrow)

            @pl.loop(0, 64, unroll=4)
            def _prep(g):
                idxa[pl.ds(g * 16, 16)] = \
                    tokrow[pl.ds(g * 16, 16)] & (_QUART - 1)

            def issue_gather(bt, slot):
                return pltpu.async_copy(
                    wtp_hbm.at[idxa.at[pl.ds(bt * 128, 128)]],
                    rows.at[slot], gsems[slot])

            g_desc = [issue_gather(j, j) for j in range(4)]
            s_desc = [None] * 4
            for bt in range(_NBT):
                slot = bt & 3
                g_desc[slot].wait()
                s_desc[slot] = pltpu.async_copy(
                    rows.at[slot], g_hbm.at[s_tok, bt], ssems[slot])
                if bt + 4 < _NBT:
                    s_desc[slot].wait()
                    g_desc[slot] = issue_gather(bt + 4, slot)
            for j in range(4):
                s_desc[j].wait()

    return sc_gather


try:
    _SC_GATHER = _make_sc_kernel()
except ValueError:
    _SC_GATHER = None


def _tc_out(g5, tok_flat, learned):
    def body(g_ref, tok_ref, lrn_ref, out_ref):
        sp = pl.program_id(0)

        @pl.when(sp < _NTOK)
        def _learned():
            row = lrn_ref[pl.ds(jnp.minimum(sp, _NTOK - 1), 1), :]
            blk = jnp.broadcast_to(
                row.reshape(1, _DIM, 1, 1), (1, _DIM, _NBT, 128))
            out_ref[...] = (blk.reshape(1, 8, 8, _NBT, 128)
                            .transpose(0, 1, 3, 2, 4))

        @pl.when(sp >= _NTOK)
        def _gather():
            ys = []
            for bt in range(_NBT):
                g = g_ref[0, bt]
                q = (tok_ref[0, bt, 0] >> 18)[:, None]
                z01 = jnp.where((q & 1) == 1, g[:, 32:64], g[:, 0:32])
                z23 = jnp.where((q & 1) == 1, g[:, 96:128], g[:, 64:96])
                z = jnp.where((q & 2) == 2, z23, z01)
                flo = lax.bitcast_convert_type(z << 16, jnp.float32)
                fhi = lax.bitcast_convert_type(
                    z & jnp.int32(-65536), jnp.float32)
                y = jnp.concatenate([flo.T, fhi.T], axis=0)
                ys.append(y.reshape(1, 8, 8, 128))
            big = jnp.concatenate(ys, axis=0)
            out_ref[...] = big.transpose(1, 0, 2, 3).reshape(1, 8, 8, 8, 128)

    return pl.pallas_call(
        body,
        grid=(_OUT_S,),
        in_specs=[
            pl.BlockSpec((1, _NBT, 128, 128),
                         lambda s: (jnp.maximum(s - _NTOK, 0), 0, 0, 0)),
            pl.BlockSpec((1, _NBT, 1, 128),
                         lambda s: (jnp.maximum(s - _NTOK, 0), 0, 0, 0)),
            pl.BlockSpec((_NTOK, _DIM), lambda s: (0, 0)),
        ],
        out_specs=pl.BlockSpec((1, 8, _NBT, 8, 128),
                               lambda s: (s, 0, 0, 0, 0)),
        out_shape=jax.ShapeDtypeStruct((_OUT_S, 8, _NBT, 8, 128),
                                       jnp.float32),
    )(g5, tok_flat, learned)


def kernel(tokens, wte_weight, learned_embedding):
    wtp = _tc_pack(wte_weight.T)
    tok2 = tokens.astype(jnp.int32).T.reshape(_SEQ, _NBT, 128)
    g5 = _SC_GATHER(wtp, tok2.reshape(-1))
    out5 = _tc_out(g5, tok2.reshape(_SEQ, _NBT, 1, 128), learned_embedding)
    return (out5.transpose(2, 4, 0, 1, 3)
            .reshape(_BATCH, _OUT_S, _DIM))
